# baseline (speedup 1.0000x reference)
import functools

import jax
import jax.numpy as jnp
from jax import lax
from jax.experimental import pallas as pl
from jax.experimental.pallas import tpu as pltpu
from jax.experimental.pallas import tpu_sc as plsc

_D = 1024
_NC = 2
_NS = 16
_NW = _NC * _NS
_CHUNK = 32
_NBUF = 3


@functools.cache
def _build(batch, seq):
    n_total = batch * seq
    b_per_w = n_total // _NW
    w_per_row = seq // b_per_w
    n_chunks = b_per_w // _CHUNK
    mesh = plsc.VectorSubcoreMesh(
        core_axis_name="c", subcore_axis_name="s",
        num_cores=_NC, num_subcores=_NS)

    @functools.partial(
        pl.kernel,
        out_type=jax.ShapeDtypeStruct((batch, seq, _D), jnp.float32),
        mesh=mesh,
        compiler_params=pltpu.CompilerParams(use_tc_tiling_on_sc=False),
        scratch_types=[
            pltpu.VMEM((n_chunks, _CHUNK), jnp.int32),
            [pltpu.VMEM((_CHUNK, _D), jnp.float32) for _ in range(_NBUF)],
            [pltpu.SemaphoreType.DMA for _ in range(_NBUF)],
            [pltpu.SemaphoreType.DMA for _ in range(_NBUF)],
        ],
    )
    def gather_kernel(idx_hbm, table_hbm, out_hbm, idx_v, bufs, sgs, sos):
        wid = lax.axis_index("s") * _NC + lax.axis_index("c")
        row = wid // w_per_row
        col = (wid % w_per_row) * b_per_w
        pltpu.sync_copy(idx_hbm.at[wid], idx_v)

        def gather(j):
            return pltpu.async_copy(
                table_hbm.at[idx_v.at[j]],
                bufs[j % _NBUF], sgs[j % _NBUF])

        def put(j):
            return pltpu.async_copy(
                bufs[j % _NBUF],
                out_hbm.at[row, pl.ds(col + j * _CHUNK, _CHUNK)],
                sos[j % _NBUF])

        gathers = [None] * n_chunks
        outs = [None] * n_chunks
        for j in range(min(_NBUF - 1, n_chunks)):
            gathers[j] = gather(j)
        for j in range(n_chunks):
            if j + _NBUF - 1 < n_chunks:
                if j >= 1:
                    outs[j - 1].wait()
                gathers[j + _NBUF - 1] = gather(j + _NBUF - 1)
            gathers[j].wait()
            outs[j] = put(j)
        for j in range(max(0, n_chunks - _NBUF), n_chunks):
            outs[j].wait()

    return gather_kernel


def kernel(positions, positional_encoding_matrix):
    b, s = positions.shape
    b_per_w = b * s // _NW
    idx3 = positions.reshape(_NW, b_per_w // _CHUNK, _CHUNK)
    return _build(b, s)(idx3, positional_encoding_matrix)


# speedup vs baseline: 2.4284x; 2.4284x over previous
import functools

import jax
import jax.numpy as jnp
from jax import lax
from jax.experimental import pallas as pl
from jax.experimental.pallas import tpu as pltpu
from jax.experimental.pallas import tpu_sc as plsc

_D = 1024
_NC = 2
_NS = 16
_NW = _NC * _NS
_CHUNK = 16
_NBUF = 6


@functools.cache
def _build(batch, seq):
    n_total = batch * seq
    b_per_w = n_total // _NW
    w_per_row = seq // b_per_w
    n_chunks = b_per_w // _CHUNK
    mesh = plsc.VectorSubcoreMesh(
        core_axis_name="c", subcore_axis_name="s",
        num_cores=_NC, num_subcores=_NS)

    @functools.partial(
        pl.kernel,
        out_type=jax.ShapeDtypeStruct((batch, seq, _D), jnp.float32),
        mesh=mesh,
        scratch_types=[
            pltpu.VMEM((b_per_w,), jnp.int32),
            [pltpu.VMEM((_CHUNK, _D), jnp.float32) for _ in range(_NBUF)],
            [pltpu.SemaphoreType.DMA for _ in range(_NBUF)],
            [pltpu.SemaphoreType.DMA for _ in range(_NBUF)],
        ],
    )
    def gather_kernel(idx_hbm, table_hbm, out_hbm, idx_v, bufs, sgs, sos):
        wid = lax.axis_index("s") * _NC + lax.axis_index("c")
        row = wid // w_per_row
        col = (wid % w_per_row) * b_per_w
        pltpu.sync_copy(idx_hbm.at[row, pl.ds(col, b_per_w)], idx_v)

        def gather(j):
            return pltpu.async_copy(
                table_hbm.at[idx_v.at[pl.ds(j * _CHUNK, _CHUNK)]],
                bufs[j % _NBUF], sgs[j % _NBUF])

        def put(j):
            return pltpu.async_copy(
                bufs[j % _NBUF],
                out_hbm.at[row, pl.ds(col + j * _CHUNK, _CHUNK)],
                sos[j % _NBUF])

        gathers = [None] * n_chunks
        outs = [None] * n_chunks
        for j in range(min(_NBUF - 1, n_chunks)):
            gathers[j] = gather(j)
        for j in range(n_chunks):
            if j + _NBUF - 1 < n_chunks:
                if j >= 1:
                    outs[j - 1].wait()
                gathers[j + _NBUF - 1] = gather(j + _NBUF - 1)
            gathers[j].wait()
            outs[j] = put(j)
        for j in range(max(0, n_chunks - _NBUF), n_chunks):
            outs[j].wait()

    return gather_kernel


def kernel(positions, positional_encoding_matrix):
    b, s = positions.shape
    return _build(b, s)(positions, positional_encoding_matrix)


# speedup vs baseline: 2.4321x; 1.0015x over previous
import functools

import jax
import jax.numpy as jnp
from jax import lax
from jax.experimental import pallas as pl
from jax.experimental.pallas import tpu as pltpu
from jax.experimental.pallas import tpu_sc as plsc

_D = 1024
_NC = 2
_NS = 16
_NW = _NC * _NS
_CHUNK = 32
_NBUF = 3


@functools.cache
def _build(batch, seq):
    n_total = batch * seq
    b_per_w = n_total // _NW
    w_per_row = seq // b_per_w
    n_chunks = b_per_w // _CHUNK
    mesh = plsc.VectorSubcoreMesh(
        core_axis_name="c", subcore_axis_name="s",
        num_cores=_NC, num_subcores=_NS)

    @functools.partial(
        pl.kernel,
        out_type=jax.ShapeDtypeStruct((batch, seq, _D), jnp.float32),
        mesh=mesh,
        scratch_types=[
            pltpu.VMEM((b_per_w,), jnp.int32),
            [pltpu.VMEM((_CHUNK, _D), jnp.float32) for _ in range(_NBUF)],
            [pltpu.SemaphoreType.DMA for _ in range(_NBUF)],
            [pltpu.SemaphoreType.DMA for _ in range(_NBUF)],
        ],
    )
    def gather_kernel(idx_hbm, table_hbm, out_hbm, idx_v, bufs, sgs, sos):
        wid = lax.axis_index("s") * _NC + lax.axis_index("c")
        row = wid // w_per_row
        col = (wid % w_per_row) * b_per_w
        pltpu.sync_copy(idx_hbm.at[row, pl.ds(col, b_per_w)], idx_v)

        def gather(j):
            return pltpu.async_copy(
                table_hbm.at[idx_v.at[pl.ds(j * _CHUNK, _CHUNK)]],
                bufs[j % _NBUF], sgs[j % _NBUF])

        def put(j):
            return pltpu.async_copy(
                bufs[j % _NBUF],
                out_hbm.at[row, pl.ds(col + j * _CHUNK, _CHUNK)],
                sos[j % _NBUF])

        gathers = [None] * n_chunks
        outs = [None] * n_chunks
        for j in range(min(_NBUF - 1, n_chunks)):
            gathers[j] = gather(j)
        for j in range(n_chunks):
            if j + _NBUF - 1 < n_chunks:
                if j >= 1:
                    outs[j - 1].wait()
                gathers[j + _NBUF - 1] = gather(j + _NBUF - 1)
            gathers[j].wait()
            outs[j] = put(j)
        for j in range(max(0, n_chunks - _NBUF), n_chunks):
            outs[j].wait()

    return gather_kernel


def kernel(positions, positional_encoding_matrix):
    b, s = positions.shape
    return _build(b, s)(positions, positional_encoding_matrix)


# speedup vs baseline: 2.4375x; 1.0023x over previous
import functools

import jax
import jax.numpy as jnp
from jax import lax
from jax.experimental import pallas as pl
from jax.experimental.pallas import tpu as pltpu
from jax.experimental.pallas import tpu_sc as plsc

_D = 1024
_NC = 2
_NS = 16
_NW = _NC * _NS
_CHUNK = 32
_NBUF = 3


@functools.cache
def _build(batch, seq):
    n_total = batch * seq
    b_per_w = n_total // _NW
    w_per_row = seq // b_per_w
    n_chunks = b_per_w // _CHUNK
    mesh = plsc.VectorSubcoreMesh(
        core_axis_name="c", subcore_axis_name="s",
        num_cores=_NC, num_subcores=_NS)

    @functools.partial(
        pl.kernel,
        out_type=jax.ShapeDtypeStruct((batch, seq, _D), jnp.float32),
        mesh=mesh,
        compiler_params=pltpu.CompilerParams(skip_device_barrier=True),
        scratch_types=[
            pltpu.VMEM((b_per_w,), jnp.int32),
            [pltpu.VMEM((_CHUNK, _D), jnp.float32) for _ in range(_NBUF)],
            [pltpu.SemaphoreType.DMA for _ in range(_NBUF)],
            [pltpu.SemaphoreType.DMA for _ in range(_NBUF)],
        ],
    )
    def gather_kernel(idx_hbm, table_hbm, out_hbm, idx_v, bufs, sgs, sos):
        wid = lax.axis_index("s") * _NC + lax.axis_index("c")
        row = wid // w_per_row
        col = (wid % w_per_row) * b_per_w
        pltpu.sync_copy(idx_hbm.at[row, pl.ds(col, b_per_w)], idx_v)

        def gather(j):
            return pltpu.async_copy(
                table_hbm.at[idx_v.at[pl.ds(j * _CHUNK, _CHUNK)]],
                bufs[j % _NBUF], sgs[j % _NBUF])

        def put(j):
            return pltpu.async_copy(
                bufs[j % _NBUF],
                out_hbm.at[row, pl.ds(col + j * _CHUNK, _CHUNK)],
                sos[j % _NBUF])

        gathers = [None] * n_chunks
        outs = [None] * n_chunks
        for j in range(min(_NBUF - 1, n_chunks)):
            gathers[j] = gather(j)
        for j in range(n_chunks):
            if j + _NBUF - 1 < n_chunks:
                if j >= 1:
                    outs[j - 1].wait()
                gathers[j + _NBUF - 1] = gather(j + _NBUF - 1)
            gathers[j].wait()
            outs[j] = put(j)
        for j in range(max(0, n_chunks - _NBUF), n_chunks):
            outs[j].wait()

    return gather_kernel


def kernel(positions, positional_encoding_matrix):
    b, s = positions.shape
    return _build(b, s)(positions, positional_encoding_matrix)
